# Initial kernel scaffold; baseline (speedup 1.0000x reference)
import jax
import jax.numpy as jnp
from jax.experimental import pallas as pl


def kernel(w1, b1, w2, b2):
    raise NotImplementedError("write your pallas kernel here")


# speedup vs baseline: 6.7520x; 6.7520x over previous
import functools

import jax
import jax.numpy as jnp
from jax.experimental import pallas as pl

_FINAL = 2049
_RES = [129, 257, 513, 1025, 2049]
_NPTS = [0, 16384, 32768, 65536, 131072]
_HALF_STEP = (1.0 / _FINAL) / 2.0


def _mlp_body(pts_ref, w1_ref, b1_ref, w2t_ref, b2_ref, o_ref):
    c = pts_ref[...] / _FINAL + _HALF_STEP
    c = c * 2.0 - 1.0
    px = c[:, 0:1]
    py = c[:, 1:2]
    h = jnp.tanh(px * w1_ref[0:1, :] + py * w1_ref[1:2, :] + b1_ref[...])
    o = jnp.sum(h * w2t_ref[...], axis=1) + b2_ref[0, 0]
    o_ref[...] = jax.nn.sigmoid(o)


def _mlp_eval(pts, w1, b1, w2, b2):
    n = pts.shape[0]
    blk = 2048
    npad = ((n + blk - 1) // blk) * blk
    pts_p = jnp.pad(pts, ((0, npad - n), (0, 0)))
    out = pl.pallas_call(
        _mlp_body,
        grid=(npad // blk,),
        in_specs=[
            pl.BlockSpec((blk, 2), lambda i: (i, 0)),
            pl.BlockSpec((2, 64), lambda i: (0, 0)),
            pl.BlockSpec((1, 64), lambda i: (0, 0)),
            pl.BlockSpec((1, 64), lambda i: (0, 0)),
            pl.BlockSpec((1, 1), lambda i: (0, 0)),
        ],
        out_specs=pl.BlockSpec((blk,), lambda i: (i,)),
        out_shape=jax.ShapeDtypeStruct((npad,), jnp.float32),
    )(pts_p, w1, b1.reshape(1, 64), w2.reshape(1, 64), b2.reshape(1, 1))
    return out[:n]


def _upsample(img, hout, wout):
    hin = img.shape[0]
    win = img.shape[1]
    ys = jnp.linspace(0.0, hin - 1.0, hout)
    xs = jnp.linspace(0.0, win - 1.0, wout)
    y0 = jnp.floor(ys).astype(jnp.int32)
    x0 = jnp.floor(xs).astype(jnp.int32)
    y1 = jnp.clip(y0 + 1, 0, hin - 1)
    x1 = jnp.clip(x0 + 1, 0, win - 1)
    wy = (ys - y0.astype(jnp.float32))[:, None]
    wx = (xs - x0.astype(jnp.float32))[None, :]
    top = img[y0, :]
    bot = img[y1, :]
    v = top * (1.0 - wy) + bot * wy
    left = v[:, x0]
    right = v[:, x1]
    return left * (1.0 - wx) + right * wx


def kernel(w1, b1, w2, b2):
    r0 = _RES[0]
    s0 = float(_FINAL - 1) / float(r0 - 1)
    ax = jnp.arange(r0, dtype=jnp.float32) * s0
    gx = jnp.tile(ax, (r0,))
    gy = jnp.repeat(ax, r0)
    pts0 = jnp.stack([gx, gy], axis=-1)
    occ = _mlp_eval(pts0, w1, b1, w2, b2).reshape(r0, r0)

    for ri in range(1, len(_RES)):
        res = _RES[ri]
        npt = _NPTS[ri]
        stride = float(_FINAL - 1) / float(res - 1)
        occ = _upsample(occ, res, res)
        flat = occ.reshape(res * res)
        unc = -jnp.abs(flat - 0.5)
        _, idx = jax.lax.top_k(unc, npt)
        px = (idx % res).astype(jnp.float32) * stride
        py = (idx // res).astype(jnp.float32) * stride
        pts = jnp.stack([px, py], axis=-1)
        vals = _mlp_eval(pts, w1, b1, w2, b2)
        flat = flat.at[idx].set(vals)
        occ = flat.reshape(res, res)

    return occ.reshape(1, 1, _FINAL, _FINAL)
